# baseline (speedup 1.0000x reference)
import jax
import jax.numpy as jnp
import numpy as np
from jax.experimental import pallas as pl

_LEVELS = [8, 8, 8, 5, 5, 5]
_NQ = 8
_EPS = 1e-3


def _fsq_tc_body(x_ref, win_ref, bin_ref, wout_ref, c_ref,
                 out_ref, idx_ref):
    hl = c_ref[:, 0:1]
    off = c_ref[:, 1:2]
    shift = c_ref[:, 2:3]
    basis = c_ref[:, 4:5]
    k0 = c_ref[0:1, 21:22]

    z = jax.lax.dot_general(
        x_ref[...], win_ref[...], (((1,), (0,)), ((), ())),
        preferred_element_type=jnp.float32)
    zT = z.T + bin_ref[...]

    r = jnp.tanh(zT + shift) * hl - off
    q = jnp.zeros_like(r)
    idx_rows = []
    for i in range(_NQ):
        inv = c_ref[:, 5 + i:6 + i]
        qs = c_ref[:, 13 + i:14 + i]
        zb = jnp.tanh(r * inv + shift) * hl - off
        rnd = jnp.round(zb)
        idxf = jnp.sum(rnd * basis, axis=0) + k0[0]
        idx_rows.append(idxf.astype(jnp.int32))
        quant = rnd * qs
        r = r - quant
        q = q + quant

    row = jax.lax.broadcasted_iota(jnp.int32, (8, 1), 0)
    q = jnp.where(row == 6, 1.0, q)
    out_ref[...] = jax.lax.dot_general(
        q, wout_ref[...], (((0,), (0,)), ((), ())),
        preferred_element_type=jnp.float32)

    idxT = jnp.stack(idx_rows, axis=0)
    idxf32 = jax.lax.bitcast_convert_type(idxT, jnp.float32)
    idx_ref[...] = jax.lax.bitcast_convert_type(idxf32.T, jnp.int32)


def kernel(x, W_in, b_in, W_out, b_out):
    B, N, D = x.shape
    T = B * N
    x2 = x.reshape(T, D)
    win8 = jnp.zeros((D, 8), jnp.float32).at[:, :6].set(W_in)
    bin8 = jnp.zeros((8, 1), jnp.float32).at[:6, 0].set(b_in)
    wout8 = jnp.zeros((8, D), jnp.float32).at[:6, :].set(W_out).at[6, :].set(b_out)

    lev = jnp.array(_LEVELS + [2, 2], dtype=jnp.float32)
    half_l = (lev - 1.0) * (1.0 + _EPS) / 2.0
    offset = jnp.where(jnp.mod(lev, 2.0) == 0.0, 0.5, 0.0)
    shift = jnp.arctanh(offset / half_l)
    hw = jnp.floor(lev / 2.0)
    basis = jnp.concatenate([
        jnp.array(np.concatenate(([1], np.cumprod(_LEVELS[:-1]))),
                  dtype=jnp.float32),
        jnp.zeros((2,), jnp.float32)])
    scales = [(lev - 1.0) ** (-float(i)) for i in range(_NQ)]
    invs = [(lev - 1.0) ** float(i) for i in range(_NQ)]
    qss = [s / hw for s in scales]
    k0 = jnp.full((8,), jnp.sum(hw * basis), jnp.float32)

    cols = [half_l, offset, shift, hw, basis]
    cols += invs
    cols += qss
    cols += [k0]
    cols += [jnp.zeros((8,), jnp.float32)] * (24 - len(cols))
    consts = jnp.stack(cols, axis=1)

    BLK = 2048
    grid = (T // BLK,)
    out, idx = pl.pallas_call(
        _fsq_tc_body,
        grid=grid,
        in_specs=[
            pl.BlockSpec((BLK, D), lambda i: (i, 0)),
            pl.BlockSpec((D, 8), lambda i: (0, 0)),
            pl.BlockSpec((8, 1), lambda i: (0, 0)),
            pl.BlockSpec((8, D), lambda i: (0, 0)),
            pl.BlockSpec((8, 24), lambda i: (0, 0)),
        ],
        out_specs=[
            pl.BlockSpec((BLK, D), lambda i: (i, 0)),
            pl.BlockSpec((BLK, 8), lambda i: (i, 0)),
        ],
        out_shape=[
            jax.ShapeDtypeStruct((T, D), jnp.float32),
            jax.ShapeDtypeStruct((T, 8), jnp.int32),
        ],
    )(x2, win8, bin8, wout8, consts)

    return out.reshape(B, N, D), idx.reshape(B, N, _NQ)


# speedup vs baseline: 1.1643x; 1.1643x over previous
import jax
import jax.numpy as jnp
import numpy as np
from jax.experimental import pallas as pl

_LEVELS = [8, 8, 8, 5, 5, 5]
_NQ = 8
_EPS = 1e-3


def _fsq_tc_body(x_ref, win_ref, bin_ref, wout_ref, c_ref,
                 out_ref, idx_ref):
    hl = c_ref[:, 0:1]
    off = c_ref[:, 1:2]
    shift = c_ref[:, 2:3]
    basis = c_ref[:, 4:5]
    k0 = c_ref[0:1, 21:22]

    z = jax.lax.dot_general(
        x_ref[...], win_ref[...], (((1,), (0,)), ((), ())),
        preferred_element_type=jnp.float32)
    zT = z.T + bin_ref[...]

    r = jnp.tanh(zT + shift) * hl - off
    q = jnp.zeros_like(r)
    idx_rows = []
    for i in range(_NQ):
        inv = c_ref[:, 5 + i:6 + i]
        qs = c_ref[:, 13 + i:14 + i]
        zb = jnp.tanh(r * inv + shift) * hl - off
        rnd = jnp.round(zb)
        idxf = jnp.sum(rnd * basis, axis=0) + k0[0]
        idx_rows.append(idxf.astype(jnp.int32))
        quant = rnd * qs
        r = r - quant
        q = q + quant

    row = jax.lax.broadcasted_iota(jnp.int32, (8, 1), 0)
    q = jnp.where(row == 6, 1.0, q)
    out_ref[...] = jax.lax.dot_general(
        q, wout_ref[...], (((0,), (0,)), ((), ())),
        preferred_element_type=jnp.float32)

    idx_ref[...] = jnp.stack(idx_rows, axis=0)


def kernel(x, W_in, b_in, W_out, b_out):
    B, N, D = x.shape
    T = B * N
    x2 = x.reshape(T, D)
    win8 = jnp.zeros((D, 8), jnp.float32).at[:, :6].set(W_in)
    bin8 = jnp.zeros((8, 1), jnp.float32).at[:6, 0].set(b_in)
    wout8 = jnp.zeros((8, D), jnp.float32).at[:6, :].set(W_out).at[6, :].set(b_out)

    lev = jnp.array(_LEVELS + [2, 2], dtype=jnp.float32)
    half_l = (lev - 1.0) * (1.0 + _EPS) / 2.0
    offset = jnp.where(jnp.mod(lev, 2.0) == 0.0, 0.5, 0.0)
    shift = jnp.arctanh(offset / half_l)
    hw = jnp.floor(lev / 2.0)
    basis = jnp.concatenate([
        jnp.array(np.concatenate(([1], np.cumprod(_LEVELS[:-1]))),
                  dtype=jnp.float32),
        jnp.zeros((2,), jnp.float32)])
    scales = [(lev - 1.0) ** (-float(i)) for i in range(_NQ)]
    invs = [(lev - 1.0) ** float(i) for i in range(_NQ)]
    qss = [s / hw for s in scales]
    k0 = jnp.full((8,), jnp.sum(hw * basis), jnp.float32)

    cols = [half_l, offset, shift, hw, basis]
    cols += invs
    cols += qss
    cols += [k0]
    cols += [jnp.zeros((8,), jnp.float32)] * (24 - len(cols))
    consts = jnp.stack(cols, axis=1)

    BLK = 2048
    grid = (T // BLK,)
    out, idxT = pl.pallas_call(
        _fsq_tc_body,
        grid=grid,
        in_specs=[
            pl.BlockSpec((BLK, D), lambda i: (i, 0)),
            pl.BlockSpec((D, 8), lambda i: (0, 0)),
            pl.BlockSpec((8, 1), lambda i: (0, 0)),
            pl.BlockSpec((8, D), lambda i: (0, 0)),
            pl.BlockSpec((8, 24), lambda i: (0, 0)),
        ],
        out_specs=[
            pl.BlockSpec((BLK, D), lambda i: (i, 0)),
            pl.BlockSpec((8, BLK), lambda i: (0, i)),
        ],
        out_shape=[
            jax.ShapeDtypeStruct((T, D), jnp.float32),
            jax.ShapeDtypeStruct((8, T), jnp.int32),
        ],
    )(x2, win8, bin8, wout8, consts)

    return out.reshape(B, N, D), idxT.T.reshape(B, N, _NQ)


# speedup vs baseline: 1.2957x; 1.1129x over previous
import jax
import jax.numpy as jnp
import numpy as np
from jax.experimental import pallas as pl
from jax.experimental.pallas import tpu as pltpu

_LEVELS = [8, 8, 8, 5, 5, 5]
_NQ = 8
_EPS = 1e-3


def _fsq_tc_body(x_ref, win_ref, bin_ref, wout_ref, c_ref,
                 out_ref, idx_ref, q_scr):
    g = pl.program_id(0)
    hl = c_ref[:, 0:1]
    off = c_ref[:, 1:2]
    shift = c_ref[:, 2:3]
    basis = c_ref[:, 4:5]
    k0 = c_ref[0:1, 21:22]
    row = jax.lax.broadcasted_iota(jnp.int32, (8, 1), 0)

    @pl.when(g > 0)
    def _drain():
        out_ref[...] = jax.lax.dot_general(
            q_scr[...], wout_ref[...], (((0,), (0,)), ((), ())),
            preferred_element_type=jnp.float32)

    z = jax.lax.dot_general(
        x_ref[...], win_ref[...], (((1,), (0,)), ((), ())),
        preferred_element_type=jnp.float32)
    zT = z.T + bin_ref[...]

    r = jnp.tanh(zT + shift) * hl - off
    q = jnp.zeros_like(r)
    idx_rows = []
    for i in range(_NQ):
        inv = c_ref[:, 5 + i:6 + i]
        qs = c_ref[:, 13 + i:14 + i]
        zb = jnp.tanh(r * inv + shift) * hl - off
        rnd = jnp.round(zb)
        idxf = jnp.sum(rnd * basis, axis=0) + k0[0]
        idx_rows.append(idxf.astype(jnp.int32))
        quant = rnd * qs
        r = r - quant
        q = q + quant

    q_scr[...] = jnp.where(row == 6, 1.0, q)
    idx_ref[...] = jnp.stack(idx_rows, axis=0)


def kernel(x, W_in, b_in, W_out, b_out):
    B, N, D = x.shape
    T = B * N
    x2 = x.reshape(T, D)
    win8 = jnp.zeros((D, 8), jnp.float32).at[:, :6].set(W_in)
    bin8 = jnp.zeros((8, 1), jnp.float32).at[:6, 0].set(b_in)
    wout8 = jnp.zeros((8, D), jnp.float32).at[:6, :].set(W_out).at[6, :].set(b_out)

    lev = jnp.array(_LEVELS + [2, 2], dtype=jnp.float32)
    half_l = (lev - 1.0) * (1.0 + _EPS) / 2.0
    offset = jnp.where(jnp.mod(lev, 2.0) == 0.0, 0.5, 0.0)
    shift = jnp.arctanh(offset / half_l)
    hw = jnp.floor(lev / 2.0)
    basis = jnp.concatenate([
        jnp.array(np.concatenate(([1], np.cumprod(_LEVELS[:-1]))),
                  dtype=jnp.float32),
        jnp.zeros((2,), jnp.float32)])
    scales = [(lev - 1.0) ** (-float(i)) for i in range(_NQ)]
    invs = [(lev - 1.0) ** float(i) for i in range(_NQ)]
    qss = [s / hw for s in scales]
    k0 = jnp.full((8,), jnp.sum(hw * basis), jnp.float32)

    cols = [half_l, offset, shift, hw, basis]
    cols += invs
    cols += qss
    cols += [k0]
    cols += [jnp.zeros((8,), jnp.float32)] * (24 - len(cols))
    consts = jnp.stack(cols, axis=1)

    BLK = 2048
    nblk = T // BLK
    grid = (nblk + 1,)
    last = nblk - 1
    out, idxT = pl.pallas_call(
        _fsq_tc_body,
        grid=grid,
        in_specs=[
            pl.BlockSpec((BLK, D), lambda i: (jnp.minimum(i, last), 0)),
            pl.BlockSpec((D, 8), lambda i: (0, 0)),
            pl.BlockSpec((8, 1), lambda i: (0, 0)),
            pl.BlockSpec((8, D), lambda i: (0, 0)),
            pl.BlockSpec((8, 24), lambda i: (0, 0)),
        ],
        out_specs=[
            pl.BlockSpec((BLK, D), lambda i: (jnp.maximum(i - 1, 0), 0)),
            pl.BlockSpec((8, BLK), lambda i: (0, jnp.minimum(i, last))),
        ],
        out_shape=[
            jax.ShapeDtypeStruct((T, D), jnp.float32),
            jax.ShapeDtypeStruct((8, T), jnp.int32),
        ],
        scratch_shapes=[pltpu.VMEM((8, BLK), jnp.float32)],
    )(x2, win8, bin8, wout8, consts)

    return out.reshape(B, N, D), idxT.T.reshape(B, N, _NQ)
